# baseline (speedup 1.0000x reference)
import functools

import jax
import jax.numpy as jnp
from jax import lax
from jax.experimental import pallas as pl
from jax.experimental.pallas import tpu as pltpu
from jax.experimental.pallas import tpu_sc as plsc

NC = 2
NS = 16
NW = NC * NS

B, L, W = 4096, 200, 32
TOTAL = B * L
SUB = 128
NSUB = 8
CHUNK = SUB * NSUB
NCHUNKS = TOTAL // CHUNK
CPW = NCHUNKS // NW
PAD_ROW = 2048


def _sc_body(idx_hbm, mask_hbm, table_hbm, out_hbm, idx_v, mask_v, idxm_v,
             rows_v, sem):
    wid = lax.axis_index("s") * NC + lax.axis_index("c")

    def chunk_body(c, carry):
        cid = wid * CPW + c
        pltpu.sync_copy(idx_hbm.at[cid], idx_v)
        pltpu.sync_copy(mask_hbm.at[cid], mask_v)
        for j in range(NSUB):
            for i in range(SUB // 16):
                sl = pl.ds(i * 16, 16)
                m = mask_v[j, sl]
                x = idx_v[j, sl]
                idxm_v[j, sl] = jnp.where(m != 0, x, PAD_ROW)
        cps = [
            pltpu.async_copy(table_hbm.at[idxm_v.at[j]], rows_v.at[j], sem)
            for j in range(NSUB)
        ]
        for cp in cps:
            cp.wait()
        pltpu.sync_copy(rows_v, out_hbm.at[cid])
        return carry

    lax.fori_loop(0, CPW, chunk_body, 0)


@jax.jit
def _run(idx3, mask3, table_pad):
    mesh = plsc.VectorSubcoreMesh(core_axis_name="c", subcore_axis_name="s")
    kfn = pl.kernel(
        _sc_body,
        out_type=jax.ShapeDtypeStruct((NCHUNKS, NSUB, SUB, W), jnp.float32),
        mesh=mesh,
        scratch_types=[
            pltpu.VMEM((NSUB, SUB), jnp.int32),
            pltpu.VMEM((NSUB, SUB), jnp.int32),
            pltpu.VMEM((NSUB, SUB), jnp.int32),
            pltpu.VMEM((NSUB, SUB, W), jnp.float32),
            pltpu.SemaphoreType.DMA,
        ],
        compiler_params=pltpu.CompilerParams(use_tc_tiling_on_sc=False),
    )
    return kfn(idx3, mask3, table_pad)


def kernel(obs_pos, obs_mask, embedding_table):
    idx3 = obs_pos.astype(jnp.int32).reshape(NCHUNKS, NSUB, SUB)
    mask3 = obs_mask.astype(jnp.int32).reshape(NCHUNKS, NSUB, SUB)
    table_pad = jnp.concatenate(
        [embedding_table, jnp.zeros((8, W), jnp.float32)], axis=0)
    out = _run(idx3, mask3, table_pad)
    return out.reshape(B, L, W)
